# baseline (speedup 1.0000x reference)
import jax
import jax.numpy as jnp
from jax.experimental import pallas as pl
from jax.experimental.pallas import tpu as pltpu

_BM = 200
_NBUF = 4
_SIZES = [_BM] * 49 + [160, 40]
_OFFS = [sum(_SIZES[:j]) for j in range(len(_SIZES))]
_TGROUPS = [(0, 3200), (3200, 6400), (6400, 9600), (9600, 10000)]


def _gcn_body(x_ref, adj_hbm, wt_ref, b_ref, out_ref, buf_ref, support_ref,
              acc_ref, sems):
    def _copy(blk, slot):
        sz = _SIZES[blk]
        return pltpu.make_async_copy(
            adj_hbm.at[pl.ds(_OFFS[blk], sz), :],
            buf_ref.at[slot, pl.ds(0, sz), :],
            sems.at[slot],
        )

    for w in range(_NBUF):
        _copy(w, w).start()

    support_ref[...] = jax.lax.dot_general(
        x_ref[...],
        wt_ref[...],
        (((1,), (1,)), ((), ())),
        preferred_element_type=jnp.float32,
    )

    for i in range(len(_SIZES)):
        slot = i % _NBUF
        sz = _SIZES[i]
        _copy(i, slot).wait()
        blk = (
            jax.lax.dot_general(
                buf_ref[slot, pl.ds(0, sz), :],
                support_ref[...],
                (((1,), (0,)), ((), ())),
                preferred_element_type=jnp.float32,
            )
            + b_ref[...]
        )
        acc_ref[pl.ds(_OFFS[i], sz), :] = blk
        if i + _NBUF < len(_SIZES):
            _copy(i + _NBUF, slot).start()
        done = _OFFS[i] + sz
        for g0, g1 in _TGROUPS[:-1]:
            if done == g1:
                out_ref[:, g0:g1] = acc_ref[g0:g1, :].T

    g0, g1 = _TGROUPS[-1]
    out_ref[:, g0:g1] = acc_ref[g0:g1, :].T


def kernel(input, adj, weight, bias):
    n, k = input.shape
    m = adj.shape[0]
    f = weight.shape[1]

    out_t = pl.pallas_call(
        _gcn_body,
        in_specs=[
            pl.BlockSpec((n, k), lambda: (0, 0)),
            pl.BlockSpec(memory_space=pl.ANY),
            pl.BlockSpec((f, k), lambda: (0, 0)),
            pl.BlockSpec((1, f), lambda: (0, 0)),
        ],
        out_specs=pl.BlockSpec((f, m), lambda: (0, 0)),
        out_shape=jax.ShapeDtypeStruct((f, m), jnp.float32),
        scratch_shapes=[
            pltpu.VMEM((_NBUF, _BM, n), jnp.float32),
            pltpu.VMEM((n, f), jnp.float32),
            pltpu.VMEM((m, f), jnp.float32),
            pltpu.SemaphoreType.DMA((_NBUF,)),
        ],
    )(input, adj, weight.T, bias.reshape(1, f))
    return out_t.T
